# baseline (speedup 1.0000x reference)
import functools

import jax
import jax.numpy as jnp
from jax import lax
from jax.experimental import pallas as pl
from jax.experimental.pallas import tpu as pltpu
from jax.experimental.pallas import tpu_sc as plsc

VOCAB_SIZE = 100
NUM_IDS = 327680
NUM_CORES = 2
NUM_SUBCORES = 16
NUM_WORKERS = NUM_CORES * NUM_SUBCORES
TOKENS_PER_WORKER = NUM_IDS // NUM_WORKERS
CHUNK_TOKENS = 256
NUM_ROUNDS = TOKENS_PER_WORKER // CHUNK_TOKENS
CHUNK_WORDS = CHUNK_TOKENS * VOCAB_SIZE
LANES = 16
UNROLL = 4
NBUF = 4
NIDX = NBUF + 1


def _sc_body(ids_hbm, out_hbm, *scratch):
    idx = list(scratch[:NIDX])
    cols = list(scratch[NIDX : NIDX + NBUF])
    sem = list(scratch[NIDX + NBUF :])

    wid = lax.axis_index("s") * NUM_CORES + lax.axis_index("c")
    tok_base = wid * TOKENS_PER_WORKER

    lane = lax.iota(jnp.int32, LANES)
    ones = jnp.full((LANES,), 1, jnp.int32)
    zeros = jnp.full((LANES,), 0, jnp.int32)

    def _zinit(buf):
        def body(i, _):
            for u in range(UNROLL):
                flat = (i * UNROLL + u) * LANES + lane
                plsc.store_scatter(
                    buf, [flat // CHUNK_TOKENS, flat % CHUNK_TOKENS], zeros
                )
            return 0

        lax.fori_loop(0, CHUNK_WORDS // (LANES * UNROLL), body, 0)

    def _scatter(buf, ids_ref, val):
        def body(j, _):
            for u in range(UNROLL):
                t = (j * UNROLL + u) * LANES
                ids16 = ids_ref[pl.ds(t, LANES)]
                plsc.store_scatter(buf, [ids16, t + lane], val)
            return 0

        lax.fori_loop(0, CHUNK_TOKENS // (LANES * UNROLL), body, 0)

    def _load_ids(r):
        tok0 = tok_base + r * CHUNK_TOKENS
        pltpu.sync_copy(ids_hbm.at[pl.ds(tok0, CHUNK_TOKENS)], idx[r % NIDX])

    def _out_slice(r):
        return out_hbm.at[:, pl.ds(tok_base + r * CHUNK_TOKENS, CHUNK_TOKENS)]

    _zinit(cols[0])
    _load_ids(0)

    pending = [None] * NBUF
    for r in range(NUM_ROUNDS):
        b = r % NBUF
        if pending[b] is not None:
            pltpu.make_async_copy(cols[b], _out_slice(pending[b]), sem[b]).wait()
            _scatter(cols[b], idx[pending[b] % NIDX], zeros)
            pending[b] = None
        elif r > 0:
            _zinit(cols[b])
        _scatter(cols[b], idx[r % NIDX], ones)
        pltpu.make_async_copy(cols[b], _out_slice(r), sem[b]).start()
        pending[b] = r
        if r + 1 < NUM_ROUNDS:
            _load_ids(r + 1)
    for b in range(NBUF):
        if pending[b] is not None:
            pltpu.make_async_copy(cols[b], _out_slice(pending[b]), sem[b]).wait()


_sc_call = functools.partial(
    pl.kernel,
    out_type=jax.ShapeDtypeStruct((VOCAB_SIZE, NUM_IDS), jnp.int32),
    mesh=plsc.VectorSubcoreMesh(core_axis_name="c", subcore_axis_name="s"),
    scratch_types=(
        [pltpu.VMEM((CHUNK_TOKENS,), jnp.int32) for _ in range(NIDX)]
        + [pltpu.VMEM((VOCAB_SIZE, CHUNK_TOKENS), jnp.int32) for _ in range(NBUF)]
        + [pltpu.SemaphoreType.DMA for _ in range(NBUF)]
    ),
    compiler_params=pltpu.CompilerParams(needs_layout_passes=False),
)(_sc_body)


def kernel(input):
    return _sc_call(input).T


# speedup vs baseline: 1.0817x; 1.0817x over previous
import functools

import jax
import jax.numpy as jnp
from jax import lax
from jax.experimental import pallas as pl
from jax.experimental.pallas import tpu as pltpu
from jax.experimental.pallas import tpu_sc as plsc

VOCAB_SIZE = 100
NUM_IDS = 327680
NUM_CORES = 2
NUM_SUBCORES = 16
NUM_WORKERS = NUM_CORES * NUM_SUBCORES
TOKENS_PER_WORKER = NUM_IDS // NUM_WORKERS
CHUNK_TOKENS = 512
NUM_ROUNDS = TOKENS_PER_WORKER // CHUNK_TOKENS
CHUNK_WORDS = CHUNK_TOKENS * VOCAB_SIZE
LANES = 16
UNROLL = 4
NBUF = 2
NIDX = NBUF + 1


def _sc_body(ids_hbm, out_hbm, *scratch):
    idx = list(scratch[:NIDX])
    cols = list(scratch[NIDX : NIDX + NBUF])
    sem = list(scratch[NIDX + NBUF :])

    wid = lax.axis_index("s") * NUM_CORES + lax.axis_index("c")
    tok_base = wid * TOKENS_PER_WORKER

    lane = lax.iota(jnp.int32, LANES)
    ones = jnp.full((LANES,), 1, jnp.int32)
    zeros = jnp.full((LANES,), 0, jnp.int32)

    def _zinit(buf):
        def body(i, _):
            for u in range(UNROLL):
                flat = (i * UNROLL + u) * LANES + lane
                plsc.store_scatter(
                    buf, [flat // CHUNK_TOKENS, flat % CHUNK_TOKENS], zeros
                )
            return 0

        lax.fori_loop(0, CHUNK_WORDS // (LANES * UNROLL), body, 0)

    def _scatter(buf, ids_ref, val):
        def body(j, _):
            for u in range(UNROLL):
                t = (j * UNROLL + u) * LANES
                ids16 = ids_ref[pl.ds(t, LANES)]
                plsc.store_scatter(buf, [ids16, t + lane], val)
            return 0

        lax.fori_loop(0, CHUNK_TOKENS // (LANES * UNROLL), body, 0)

    def _load_ids(r):
        tok0 = tok_base + r * CHUNK_TOKENS
        pltpu.sync_copy(ids_hbm.at[pl.ds(tok0, CHUNK_TOKENS)], idx[r % NIDX])

    def _out_slice(r):
        return out_hbm.at[:, pl.ds(tok_base + r * CHUNK_TOKENS, CHUNK_TOKENS)]

    _zinit(cols[0])
    _load_ids(0)

    pending = [None] * NBUF
    for r in range(NUM_ROUNDS):
        b = r % NBUF
        if pending[b] is not None:
            pltpu.make_async_copy(cols[b], _out_slice(pending[b]), sem[b]).wait()
            _scatter(cols[b], idx[pending[b] % NIDX], zeros)
            pending[b] = None
        elif r > 0:
            _zinit(cols[b])
        _scatter(cols[b], idx[r % NIDX], ones)
        pltpu.make_async_copy(cols[b], _out_slice(r), sem[b]).start()
        pending[b] = r
        if r + 1 < NUM_ROUNDS:
            _load_ids(r + 1)
    for b in range(NBUF):
        if pending[b] is not None:
            pltpu.make_async_copy(cols[b], _out_slice(pending[b]), sem[b]).wait()


_sc_call = functools.partial(
    pl.kernel,
    out_type=jax.ShapeDtypeStruct((VOCAB_SIZE, NUM_IDS), jnp.int32),
    mesh=plsc.VectorSubcoreMesh(core_axis_name="c", subcore_axis_name="s"),
    scratch_types=(
        [pltpu.VMEM((CHUNK_TOKENS,), jnp.int32) for _ in range(NIDX)]
        + [pltpu.VMEM((VOCAB_SIZE, CHUNK_TOKENS), jnp.int32) for _ in range(NBUF)]
        + [pltpu.SemaphoreType.DMA for _ in range(NBUF)]
    ),
    compiler_params=pltpu.CompilerParams(needs_layout_passes=False),
)(_sc_body)


def kernel(input):
    return _sc_call(input).T
